# baseline (speedup 1.0000x reference)
import functools

import jax
import jax.numpy as jnp
from jax import lax
from jax.experimental import pallas as pl
from jax.experimental.pallas import tpu as pltpu
from jax.experimental.pallas import tpu_sc as plsc

V = 1000
VP = 1024
B = 1024
S = 50
N = B * S
NW = 32
R = N // NW
SP = 56
NBATCH = R // S
NSUB = SP // 8
L = 16


def _logz_body(bigram_ref, out_ref):
    x = bigram_ref[...]
    m = jnp.max(x, axis=1, keepdims=True)
    s = jnp.sum(jnp.exp(x - m), axis=1, keepdims=True)
    out_ref[...] = (m + jnp.log(s)).reshape(1, V)


def _compute_logz(bigram):
    return pl.pallas_call(
        _logz_body,
        out_shape=jax.ShapeDtypeStruct((1, V), jnp.float32),
    )(bigram)


def _finalize_body(part_ref, out_ref):
    out_ref[...] = jnp.sum(part_ref[...], keepdims=True).reshape(1, 1) * (1.0 / N)


def _finalize(partials):
    return pl.pallas_call(
        _finalize_body,
        out_shape=jax.ShapeDtypeStruct((1, 1), jnp.float32),
    )(partials)


def _sc_body(tpad_hbm, flat_hbm, idxp_hbm, idx_hbm, tgt_hbm, logz_hbm,
             y_hbm, part_hbm,
             idx_v, tgt_v, fidx_v, lz_v, pick_v, idxb_v, bufs, cbuf, acc_v,
             sem_g, sem_p):
    wid = lax.axis_index("s") * 2 + lax.axis_index("c")
    base = wid * R
    pltpu.sync_copy(idx_hbm.at[pl.ds(base, R)], idx_v)
    pltpu.sync_copy(tgt_hbm.at[pl.ds(base, R)], tgt_v)

    lz_copy = pltpu.async_copy(logz_hbm.at[idx_v], lz_v, sem_p)

    def mkflat(g, _):
        o = pl.ds(g * L, L)
        fidx_v[o] = idx_v[o] * V + tgt_v[o]
        return 0

    lax.fori_loop(0, R // L, mkflat, 0)
    lz_copy.wait()
    pltpu.async_copy(flat_hbm.at[fidx_v], pick_v, sem_p).wait()

    def accum(g, acc):
        o = pl.ds(g * L, L)
        return acc + (lz_v[o] - pick_v[o])

    acc = lax.fori_loop(0, R // L, accum, jnp.zeros((L,), jnp.float32))
    acc_v[...] = acc
    pltpu.sync_copy(acc_v, part_hbm.at[pl.ds(wid * L, L)])

    bbase = wid * NBATCH

    def compact_row(src_r, dst_r):
        for k in range(V // L):
            cbuf[dst_r, pl.ds(k * L, L)] = bufs[0, src_r, pl.ds(k * L, L)]
        cbuf[dst_r, pl.ds(V - L, L)] = bufs[0, src_r, pl.ds(V - L, L)]

    def batch(bl, _):
        b = bbase + bl
        pltpu.sync_copy(idxp_hbm.at[pl.ds(b * SP, SP)], idxb_v)
        for j in range(NSUB):
            pltpu.async_copy(tpad_hbm.at[idxb_v.at[pl.ds(j * 8, 8)]],
                             bufs.at[0], sem_g).wait()
            nr = 8 if j < NSUB - 1 else S - 8 * (NSUB - 1)
            for r in range(nr):
                compact_row(r, j * 8 + r)
        pltpu.sync_copy(cbuf, y_hbm.at[b])
        return 0

    lax.fori_loop(0, NBATCH, batch, 0)


@jax.jit
def kernel(idx, target, bigram):
    idx_f = idx.reshape(N)
    tgt_f = target.reshape(N)
    idx_p = jnp.pad(idx, ((0, 0), (0, SP - S))).reshape(B * SP)
    flat = bigram.reshape(V * V)
    tpad = jnp.pad(bigram, ((0, 0), (0, VP - V)))
    logz = _compute_logz(bigram).reshape(V)

    mesh = plsc.VectorSubcoreMesh(core_axis_name="c", subcore_axis_name="s")
    sc = functools.partial(
        pl.kernel,
        mesh=mesh,
        out_type=[
            jax.ShapeDtypeStruct((B, S, V), jnp.float32),
            jax.ShapeDtypeStruct((NW * L,), jnp.float32),
        ],
        scratch_types=[
            pltpu.VMEM((R,), jnp.int32),
            pltpu.VMEM((R,), jnp.int32),
            pltpu.VMEM((R,), jnp.int32),
            pltpu.VMEM((R,), jnp.float32),
            pltpu.VMEM((R,), jnp.float32),
            pltpu.VMEM((SP,), jnp.int32),
            pltpu.VMEM((1, 8, VP), jnp.float32),
            pltpu.VMEM((S, V), jnp.float32),
            pltpu.VMEM((L,), jnp.float32),
            pltpu.SemaphoreType.DMA,
            pltpu.SemaphoreType.DMA,
        ],
    )(_sc_body)
    y, partials = sc(tpad, flat, idx_p, idx_f, tgt_f, logz)

    loss = _finalize(partials.reshape(1, NW * L))
    return y, loss.reshape(())


# speedup vs baseline: 1.7464x; 1.7464x over previous
import functools

import jax
import jax.numpy as jnp
from jax import lax
from jax.experimental import pallas as pl
from jax.experimental.pallas import tpu as pltpu
from jax.experimental.pallas import tpu_sc as plsc

V = 1000
VP = 1024
B = 1024
S = 50
N = B * S
NW = 32
R = N // NW
C = 40
NCHUNK = R // C
L = 16


def _logz_body(bigram_ref, out_ref):
    x = bigram_ref[...]
    m = jnp.max(x, axis=1, keepdims=True)
    s = jnp.sum(jnp.exp(x - m), axis=1, keepdims=True)
    out_ref[...] = (m + jnp.log(s)).reshape(1, V)


def _compute_logz(bigram):
    return pl.pallas_call(
        _logz_body,
        out_shape=jax.ShapeDtypeStruct((1, V), jnp.float32),
    )(bigram)


def _finalize_body(part_ref, out_ref):
    out_ref[...] = jnp.sum(part_ref[...], keepdims=True).reshape(1, 1) * (1.0 / N)


def _finalize(partials):
    return pl.pallas_call(
        _finalize_body,
        out_shape=jax.ShapeDtypeStruct((1, 1), jnp.float32),
    )(partials)


def _sc_body(tpad_hbm, flat_hbm, idx_hbm, tgt_hbm, logz_hbm,
             y_hbm, part_hbm,
             idx_v, tgt_v, fidx_v, lz_v, pick_v, bufs, acc_v,
             sem_g, sem_o, sem_p):
    wid = lax.axis_index("s") * 2 + lax.axis_index("c")
    base = wid * R
    pltpu.sync_copy(idx_hbm.at[pl.ds(base, R)], idx_v)
    pltpu.sync_copy(tgt_hbm.at[pl.ds(base, R)], tgt_v)

    lz_copy = pltpu.async_copy(logz_hbm.at[idx_v], lz_v, sem_p)

    def mkflat(g, _):
        o = pl.ds(g * L, L)
        fidx_v[o] = idx_v[o] * V + tgt_v[o]
        return 0

    lax.fori_loop(0, R // L, mkflat, 0)
    lz_copy.wait()
    pltpu.async_copy(flat_hbm.at[fidx_v], pick_v, sem_p).wait()

    def accum(g, acc):
        o = pl.ds(g * L, L)
        return acc + (lz_v[o] - pick_v[o])

    acc = lax.fori_loop(0, R // L, accum, jnp.zeros((L,), jnp.float32))
    acc_v[...] = acc
    pltpu.sync_copy(acc_v, part_hbm.at[pl.ds(wid * L, L)])

    def gather_start(c, slot):
        return pltpu.async_copy(tpad_hbm.at[idx_v.at[pl.ds(c * C, C)]],
                                bufs.at[slot], sem_g)

    def gather_wait(slot):
        pltpu.make_async_copy(tpad_hbm.at[pl.ds(0, C)], bufs.at[slot],
                              sem_g).wait()

    def out_start(c, slot):
        return pltpu.async_copy(bufs.at[slot],
                                y_hbm.at[pl.ds(base + c * C, C), :], sem_o)

    def out_wait(c, slot):
        pltpu.make_async_copy(bufs.at[slot],
                              y_hbm.at[pl.ds(base + c * C, C), :],
                              sem_o).wait()

    gather_start(0, 0)

    def chunk(c, _):
        slot = c % 2
        gather_wait(slot)

        @pl.when(c >= 1)
        def _():
            out_wait(c - 1, 1 - slot)

        @pl.when(c + 1 < NCHUNK)
        def _():
            gather_start(c + 1, 1 - slot)

        out_start(c, slot)
        return 0

    lax.fori_loop(0, NCHUNK, chunk, 0)
    out_wait(NCHUNK - 1, (NCHUNK - 1) % 2)


@jax.jit
def kernel(idx, target, bigram):
    idx_f = idx.reshape(N)
    tgt_f = target.reshape(N)
    flat = bigram.reshape(V * V)
    tpad = jnp.pad(bigram, ((0, 0), (0, VP - V)))
    logz = _compute_logz(bigram).reshape(V)

    mesh = plsc.VectorSubcoreMesh(core_axis_name="c", subcore_axis_name="s")
    sc = functools.partial(
        pl.kernel,
        mesh=mesh,
        out_type=[
            jax.ShapeDtypeStruct((N, VP), jnp.float32),
            jax.ShapeDtypeStruct((NW * L,), jnp.float32),
        ],
        scratch_types=[
            pltpu.VMEM((R,), jnp.int32),
            pltpu.VMEM((R,), jnp.int32),
            pltpu.VMEM((R,), jnp.int32),
            pltpu.VMEM((R,), jnp.float32),
            pltpu.VMEM((R,), jnp.float32),
            pltpu.VMEM((2, C, VP), jnp.float32),
            pltpu.VMEM((L,), jnp.float32),
            pltpu.SemaphoreType.DMA,
            pltpu.SemaphoreType.DMA,
            pltpu.SemaphoreType.DMA,
        ],
    )(_sc_body)
    y_pad, partials = sc(tpad, flat, idx_f, tgt_f, logz)

    loss = _finalize(partials.reshape(1, NW * L))
    return y_pad[:, :V].reshape(B, S, V), loss.reshape(())
